# Initial kernel scaffold; baseline (speedup 1.0000x reference)
import jax
import jax.numpy as jnp
from jax.experimental import pallas as pl


def kernel(features, edges, edges2, edge_features, additional_feature, W_l, b_l, W_r, W1, b1, W2, b2, gamma, beta, run_mean, run_var):
    raise NotImplementedError("write your pallas kernel here")


# speedup vs baseline: 6.8096x; 6.8096x over previous
import functools

import jax
import jax.numpy as jnp
from jax import lax
from jax.experimental import pallas as pl
from jax.experimental.pallas import tpu as pltpu
from jax.experimental.pallas import tpu_sc as plsc

N = 10000
NP = 10240
E = 320000
D = 128
CH = 128
NCH = E // CH
NC = 2
NS = 16
NW = NC * NS
RPT = NP // NS
ZR = 128
DR = NP // D


def _segment_sum_sc(features, src2d, dst2d):
  mesh = plsc.VectorSubcoreMesh(
      core_axis_name="c", subcore_axis_name="s", num_cores=NC, num_subcores=NS)

  @functools.partial(
      pl.kernel,
      out_type=(jax.ShapeDtypeStruct((NC, NP, D), jnp.float32),
                jax.ShapeDtypeStruct((NC, DR, D), jnp.float32)),
      mesh=mesh,
      compiler_params=pltpu.CompilerParams(needs_layout_passes=False),
      scratch_types=[
          pltpu.VMEM((CH,), jnp.int32),
          pltpu.VMEM((CH,), jnp.int32),
          pltpu.VMEM((CH, D), jnp.float32),
          pltpu.VMEM((ZR, D), jnp.float32),
          pltpu.VMEM((DR, D), jnp.float32),
          pltpu.VMEM((DR,), jnp.int32),
          pltpu.VMEM_SHARED((NP, D), jnp.float32),
          pltpu.VMEM_SHARED((DR, D), jnp.float32),
          pltpu.SemaphoreType.DMA,
      ],
  )
  def sage_kernel(feat_hbm, src_hbm, dst_hbm, out_hbm, deg_hbm,
                  sidx, didx, rows, zbuf, degbuf, rowid, aggsh, degsh, sem):
    c = lax.axis_index("c")
    s = lax.axis_index("s")
    wid = s * NC + c

    zeros16 = jnp.zeros((16,), jnp.float32)

    def zfill(i, _):
      for j in range(D // 16):
        zbuf[i, pl.ds(j * 16, 16)] = zeros16
      return 0

    def dzfill(i, _):
      for j in range(D // 16):
        degbuf[i, pl.ds(j * 16, 16)] = zeros16
      return 0

    lax.fori_loop(0, ZR, zfill, 0)
    lax.fori_loop(0, DR, dzfill, 0)
    for t in range(DR // 16):
      rowid[pl.ds(t * 16, 16)] = lax.iota(jnp.int32, 16) + (t * 16)

    base_row = s * RPT
    for r in range(RPT // ZR):
      pltpu.sync_copy(zbuf, aggsh.at[pl.ds(base_row + r * ZR, ZR)])

    @pl.when(s == 0)
    def _():
      pltpu.sync_copy(zbuf.at[pl.ds(0, DR)], degsh)

    plsc.subcore_barrier()

    nch = 78 + (wid < 4).astype(jnp.int32)
    start = wid * 78 + jnp.minimum(wid, 4)

    def body(i, _):
      ci = start + i
      pltpu.sync_copy(src_hbm.at[ci], sidx)
      pltpu.sync_copy(dst_hbm.at[ci], didx)
      pltpu.async_copy(feat_hbm.at[sidx], rows, sem).wait()
      pltpu.sync_copy(rows, aggsh.at[didx], add=True)
      for k in range(CH // 16):
        v = didx[pl.ds(k * 16, 16)]
        cnt, last = plsc.scan_count(v)
        plsc.addupdate_scatter(
            degbuf, [lax.shift_right_logical(v, 7), lax.bitwise_and(v, 127)],
            cnt.astype(jnp.float32), mask=last)
      return 0

    lax.fori_loop(0, nch, body, 0)

    pltpu.sync_copy(degbuf, degsh.at[rowid], add=True)
    plsc.subcore_barrier()

    for r in range(RPT // ZR):
      row0 = base_row + r * ZR
      pltpu.sync_copy(aggsh.at[pl.ds(row0, ZR)], out_hbm.at[c, pl.ds(row0, ZR)])

    @pl.when(s == 0)
    def _():
      pltpu.sync_copy(degsh, deg_hbm.at[c])

  return sage_kernel(features, src2d, dst2d)


def _mlp_body(p_ref, d_ref, f_ref, a_ref, wlt, wrt, bl, w1a, w1b, b1, w2f, b2f,
              o_ref):
  agg = p_ref[0] + p_ref[1]
  deg = d_ref[0] + d_ref[1]
  mean = agg / jnp.maximum(deg, 1.0)
  x = (jnp.dot(mean, wlt[...], preferred_element_type=jnp.float32,
               precision=lax.Precision.HIGHEST)
       + jnp.dot(f_ref[...], wrt[...], preferred_element_type=jnp.float32,
                 precision=lax.Precision.HIGHEST)
       + bl[...])
  x = jnp.where(x >= 0, x, 0.01 * x)
  h = (jnp.dot(x, w1a[...], preferred_element_type=jnp.float32,
               precision=lax.Precision.HIGHEST)
       + jnp.dot(a_ref[...], w1b[...], preferred_element_type=jnp.float32,
                 precision=lax.Precision.HIGHEST)
       + b1[...])
  h = jnp.maximum(h, 0.0)
  o_ref[...] = jnp.dot(h, w2f[...], preferred_element_type=jnp.float32,
                       precision=lax.Precision.HIGHEST) + b2f[...]


def _mlp_tc(parts, degp, features, additional,
            wlt, wrt, bl, w1a, w1b, b1, w2f, b2f):
  R = 1024
  grid = (NP // R,)
  full = lambda shape: pl.BlockSpec(shape, lambda i: (0,) * len(shape))
  return pl.pallas_call(
      _mlp_body,
      grid=grid,
      in_specs=[
          pl.BlockSpec((NC, R, D), lambda i: (0, i, 0)),
          pl.BlockSpec((NC, R, 1), lambda i: (0, i, 0)),
          pl.BlockSpec((R, D), lambda i: (i, 0)),
          pl.BlockSpec((R, 20), lambda i: (i, 0)),
          full((D, D)), full((D, D)), full((1, D)),
          full((D, 37)), full((20, 37)), full((1, 37)),
          full((37, 3)), full((1, 3)),
      ],
      out_specs=pl.BlockSpec((R, 3), lambda i: (i, 0)),
      out_shape=jax.ShapeDtypeStruct((N, 3), jnp.float32),
  )(parts, degp, features, additional, wlt, wrt, bl, w1a, w1b, b1, w2f, b2f)


def kernel(features, edges, edges2, edge_features, additional_feature,
           W_l, b_l, W_r, W1, b1, W2, b2, gamma, beta, run_mean, run_var):
  src2d = edges[0].reshape(NCH, CH)
  dst2d = edges[1].reshape(NCH, CH)
  parts, degparts = _segment_sum_sc(features, src2d, dst2d)
  degp = degparts.reshape(NC, NP, 1)

  scale = gamma / jnp.sqrt(run_var + 1e-5)
  shift = beta - run_mean * scale
  w2f = (W2 * scale[None, :]).T
  b2f = b2 + shift @ W2.T

  return _mlp_tc(
      parts, degp, features, additional_feature,
      W_l.T, W_r.T, b_l[None, :],
      W1[:, :D].T, W1[:, D:].T, b1[None, :],
      w2f, b2f[None, :])
